# baseline (speedup 1.0000x reference)
import functools

import jax
import jax.numpy as jnp
from jax import lax
from jax.experimental import pallas as pl
from jax.experimental.pallas import tpu as pltpu
from jax.experimental.pallas import tpu_sc as plsc

NC = 2
NS = 16
LANES = 16
NW = NC * NS
NPIECE = 8


def _align8(n):
    return ((n + 7) // 8) * 8


def _table_layout(E):
    t3 = E * E * E
    poff = _align8(t3)
    soff = _align8(poff + E * E)
    stride = _align8(soff + E)
    return poff, soff, stride


@functools.partial(jax.jit, static_argnums=(2, 3, 4, 5))
def _embedding_bag_mean(x_words, tbl_flat, B, L, E, D):
    chunk = B // NW
    nwords = L // 4
    nblk = nwords // 3
    POFF, SOFF, STRIDE = _table_layout(E)

    nrem_w = nwords % 3
    rem_bytes = [(3 * nblk + w, j) for w in range(nrem_w) for j in range(4)]
    rem_trips = [tuple(rem_bytes[i: i + 3])
                 for i in range(0, len(rem_bytes) - 2, 3)]
    rem2 = rem_bytes[3 * len(rem_trips):]
    rem_pairs = [tuple(rem2[i: i + 2]) for i in range(0, len(rem2) - 1, 2)]
    rem_sing = rem2[2 * len(rem_pairs):]

    mesh = plsc.VectorSubcoreMesh(core_axis_name="c", subcore_axis_name="s")

    @functools.partial(
        pl.kernel,
        out_type=jax.ShapeDtypeStruct((B * D,), jnp.float32),
        mesh=mesh,
        scratch_types=[
            pltpu.VMEM((chunk * nwords,), jnp.int32),
            pltpu.VMEM((chunk * D,), jnp.float32),
            pltpu.VMEM((D * STRIDE,), jnp.float32),
            pltpu.SemaphoreType.DMA,
            pltpu.SemaphoreType.DMA,
        ],
        compiler_params=pltpu.CompilerParams(needs_layout_passes=False),
    )
    def sc_kernel(x_hbm, tbl_hbm, out_hbm, x_v, out_v, tbl_v, sem_t, sem_x):
        wid = lax.axis_index("s") * NC + lax.axis_index("c")
        base = wid * chunk
        wbase = base * nwords
        pw = chunk * nwords // NPIECE

        tcopy = pltpu.make_async_copy(tbl_hbm, tbl_v, sem_t)
        tcopy.start()
        pieces = [
            pltpu.make_async_copy(
                x_hbm.at[pl.ds(wbase + r * pw, pw)],
                x_v.at[pl.ds(r * pw, pw)],
                sem_x,
            )
            for r in range(NPIECE)
        ]
        for c in pieces:
            c.start()
        tcopy.wait()

        tsub = [tbl_v.at[pl.ds(d * STRIDE, E * E * E)] for d in range(D)]
        psub = [tbl_v.at[pl.ds(d * STRIDE + POFF, E * E)] for d in range(D)]
        ssub = [tbl_v.at[pl.ds(d * STRIDE + SOFF, E)] for d in range(D)]

        lane = lax.iota(jnp.int32, LANES)
        scale = jnp.float32(1.0 / L)
        e_vec = jnp.full((LANES,), E, jnp.int32)
        gpp = chunk // NPIECE // LANES

        def peel(word, j):
            sh = 8 * j
            return word >> 24 if sh == 24 else (word >> sh) & 255

        def group_body(g, _):
            rows = g * LANES + lane
            rows_w = rows * nwords

            def blk_body(k, accs):
                accs = list(accs)
                wb = 3 * k
                w0 = plsc.load_gather(x_v, [rows_w + wb])
                w1 = plsc.load_gather(x_v, [rows_w + (wb + 1)])
                w2 = plsc.load_gather(x_v, [rows_w + (wb + 2)])
                byts = [peel(w, j) for w in (w0, w1, w2) for j in range(4)]
                for t in range(4):
                    tidx = (byts[3 * t] * e_vec + byts[3 * t + 1]) * e_vec \
                        + byts[3 * t + 2]
                    for d in range(D):
                        accs[d] = accs[d] + plsc.load_gather(tsub[d], [tidx])
                return tuple(accs)

            accs = lax.fori_loop(
                0, nblk, blk_body,
                tuple(jnp.zeros((LANES,), jnp.float32) for _ in range(D)),
            )
            accs = list(accs)

            if rem_bytes:
                wv = {
                    wd: plsc.load_gather(x_v, [rows_w + wd])
                    for wd in sorted({w for (w, _) in rem_bytes})
                }
                for (a, b, c) in rem_trips:
                    tidx = (peel(wv[a[0]], a[1]) * e_vec
                            + peel(wv[b[0]], b[1])) * e_vec \
                        + peel(wv[c[0]], c[1])
                    for d in range(D):
                        accs[d] = accs[d] + plsc.load_gather(tsub[d], [tidx])
                for (a, b) in rem_pairs:
                    pidx = peel(wv[a[0]], a[1]) * e_vec + peel(wv[b[0]], b[1])
                    for d in range(D):
                        accs[d] = accs[d] + plsc.load_gather(psub[d], [pidx])
                for a in rem_sing:
                    sidx = peel(wv[a[0]], a[1])
                    for d in range(D):
                        accs[d] = accs[d] + plsc.load_gather(ssub[d], [sidx])

            out_base = rows * D
            for d in range(D):
                plsc.store_scatter(out_v, [out_base + d], accs[d] * scale)
            return 0

        for r in range(NPIECE):
            pieces[r].wait()
            lax.fori_loop(r * gpp, (r + 1) * gpp, group_body, 0)

        pltpu.sync_copy(out_v, out_hbm.at[pl.ds(base * D, chunk * D)])

    return sc_kernel(x_words, tbl_flat)


def _pack_words(x, B, L):
    lq = L // 4
    rb = 512

    def pack_body(x_ref, out_ref):
        xb = x_ref[...]
        wds = (
            xb[:, 0:lq] + xb[:, lq: 2 * lq] * 256
            + xb[:, 2 * lq: 3 * lq] * 65536 + xb[:, 3 * lq:] * 16777216
        )
        out_ref[...] = wds

    packed = pl.pallas_call(
        pack_body,
        grid=(B // rb,),
        in_specs=[pl.BlockSpec((rb, L), lambda i: (i, 0))],
        out_specs=pl.BlockSpec((rb, lq), lambda i: (i, 0)),
        out_shape=jax.ShapeDtypeStruct((B, lq), jnp.int32),
    )(x)
    return packed.reshape(B * lq)


def kernel(x, weight):
    B, L = x.shape
    E, D = weight.shape
    x_words = _pack_words(x.astype(jnp.int32), B, L)
    w = weight.astype(jnp.float32)
    poff, soff, stride = _table_layout(E)
    pairs = (w[:, None, :] + w[None, :, :]).reshape(E * E, D)
    trips = (pairs[:, None, :] + w[None, :, :]).reshape(E * E * E, D)
    tbl = (
        jnp.zeros((D, stride), jnp.float32)
        .at[:, : E * E * E].set(trips.T)
        .at[:, poff: poff + E * E].set(pairs.T)
        .at[:, soff: soff + E].set(w.T)
        .reshape(-1)
    )
    out = _embedding_bag_mean(x_words, tbl, B, L, E, D)
    return out.reshape(B, D)


# speedup vs baseline: 1.2515x; 1.2515x over previous
import functools

import numpy as np

import jax
import jax.numpy as jnp
from jax import lax
from jax.experimental import pallas as pl
from jax.experimental.pallas import tpu as pltpu
from jax.experimental.pallas import tpu_sc as plsc

NC = 2
NS = 16
LANES = 16
NW = NC * NS
NPIECE = 8


def _align8(n):
    return ((n + 7) // 8) * 8


def _table_layout(E):
    t3 = E * E * E
    poff = _align8(t3)
    soff = _align8(poff + E * E)
    stride = _align8(soff + E)
    return poff, soff, stride


@functools.partial(jax.jit, static_argnums=(2, 3, 4, 5))
def _embedding_bag_mean(x_words, tbl_flat, B, L, E, D):
    chunk = B // NW
    nwords = L // 4
    nblk = nwords // 3
    POFF, SOFF, STRIDE = _table_layout(E)

    nrem_w = nwords % 3
    rem_bytes = [(3 * nblk + w, j) for w in range(nrem_w) for j in range(4)]
    rem_trips = [tuple(rem_bytes[i: i + 3])
                 for i in range(0, len(rem_bytes) - 2, 3)]
    rem2 = rem_bytes[3 * len(rem_trips):]
    rem_pairs = [tuple(rem2[i: i + 2]) for i in range(0, len(rem2) - 1, 2)]
    rem_sing = rem2[2 * len(rem_pairs):]

    mesh = plsc.VectorSubcoreMesh(core_axis_name="c", subcore_axis_name="s")

    @functools.partial(
        pl.kernel,
        out_type=jax.ShapeDtypeStruct((B * D,), jnp.float32),
        mesh=mesh,
        scratch_types=[
            pltpu.VMEM((chunk * nwords,), jnp.int32),
            pltpu.VMEM((chunk * D,), jnp.float32),
            pltpu.VMEM((D * STRIDE,), jnp.float32),
            pltpu.SemaphoreType.DMA,
            pltpu.SemaphoreType.DMA,
        ],
        compiler_params=pltpu.CompilerParams(needs_layout_passes=False),
    )
    def sc_kernel(x_hbm, tbl_hbm, out_hbm, x_v, out_v, tbl_v, sem_t, sem_x):
        wid = lax.axis_index("s") * NC + lax.axis_index("c")
        base = wid * chunk
        wbase = base * nwords
        pw = chunk * nwords // NPIECE

        tcopy = pltpu.make_async_copy(tbl_hbm, tbl_v, sem_t)
        tcopy.start()
        pieces = [
            pltpu.make_async_copy(
                x_hbm.at[pl.ds(wbase + r * pw, pw)],
                x_v.at[pl.ds(r * pw, pw)],
                sem_x,
            )
            for r in range(NPIECE)
        ]
        for c in pieces:
            c.start()
        tcopy.wait()

        tsub = [tbl_v.at[pl.ds(d * STRIDE, E * E * E)] for d in range(D)]
        psub = [tbl_v.at[pl.ds(d * STRIDE + POFF, E * E)] for d in range(D)]
        ssub = [tbl_v.at[pl.ds(d * STRIDE + SOFF, E)] for d in range(D)]

        lane = lax.iota(jnp.int32, LANES)
        scale = jnp.float32(1.0 / L)
        e_vec = jnp.full((LANES,), E, jnp.int32)
        gpp = chunk // NPIECE // LANES

        def peel(word, j):
            sh = 8 * j
            return word >> 24 if sh == 24 else (word >> sh) & 255

        def group_body(g, _):
            rows = g * LANES + lane
            rows_w = rows * nwords

            def blk_body(k, accs):
                accs = list(accs)
                wb = 3 * k
                w0 = plsc.load_gather(x_v, [rows_w + wb])
                w1 = plsc.load_gather(x_v, [rows_w + (wb + 1)])
                w2 = plsc.load_gather(x_v, [rows_w + (wb + 2)])
                byts = [peel(w, j) for w in (w0, w1, w2) for j in range(4)]
                for t in range(4):
                    tidx = (byts[3 * t] * e_vec + byts[3 * t + 1]) * e_vec \
                        + byts[3 * t + 2]
                    for d in range(D):
                        accs[d] = accs[d] + plsc.load_gather(tsub[d], [tidx])
                return tuple(accs)

            accs = lax.fori_loop(
                0, nblk, blk_body,
                tuple(jnp.zeros((LANES,), jnp.float32) for _ in range(D)),
            )
            accs = list(accs)

            if rem_bytes:
                wv = {
                    wd: plsc.load_gather(x_v, [rows_w + wd])
                    for wd in sorted({w for (w, _) in rem_bytes})
                }
                for (a, b, c) in rem_trips:
                    tidx = (peel(wv[a[0]], a[1]) * e_vec
                            + peel(wv[b[0]], b[1])) * e_vec \
                        + peel(wv[c[0]], c[1])
                    for d in range(D):
                        accs[d] = accs[d] + plsc.load_gather(tsub[d], [tidx])
                for (a, b) in rem_pairs:
                    pidx = peel(wv[a[0]], a[1]) * e_vec + peel(wv[b[0]], b[1])
                    for d in range(D):
                        accs[d] = accs[d] + plsc.load_gather(psub[d], [pidx])
                for a in rem_sing:
                    sidx = peel(wv[a[0]], a[1])
                    for d in range(D):
                        accs[d] = accs[d] + plsc.load_gather(ssub[d], [sidx])

            out_base = rows * D
            for d in range(D):
                plsc.store_scatter(out_v, [out_base + d], accs[d] * scale)
            return 0

        for r in range(NPIECE):
            pieces[r].wait()
            lax.fori_loop(r * gpp, (r + 1) * gpp, group_body, 0)

        pltpu.sync_copy(out_v, out_hbm.at[pl.ds(base * D, chunk * D)])

    return sc_kernel(x_words, tbl_flat)


def _table_mask(E):
    poff, soff, stride = _table_layout(E)
    M = np.zeros((E, stride), np.float32)
    for i in range(E):
        for j in range(E):
            for k in range(E):
                r = (i * E + j) * E + k
                M[i, r] += 1.0
                M[j, r] += 1.0
                M[k, r] += 1.0
            r = poff + i * E + j
            M[i, r] += 1.0
            M[j, r] += 1.0
        M[i, soff + i] += 1.0
    return M


def kernel(x, weight):
    B, L = x.shape
    E, D = weight.shape
    xr = x.astype(jnp.int32).reshape(B, 4, L // 4)
    x_words = (
        xr[:, 0, :] + xr[:, 1, :] * 256
        + xr[:, 2, :] * 65536 + xr[:, 3, :] * 16777216
    ).reshape(B * (L // 4))
    w = weight.astype(jnp.float32)
    tbl = jnp.einsum("ed,es->ds", w, jnp.asarray(_table_mask(E))).reshape(-1)
    out = _embedding_bag_mean(x_words, tbl, B, L, E, D)
    return out.reshape(B, D)


# speedup vs baseline: 1.3417x; 1.0721x over previous
import functools

import numpy as np

import jax
import jax.numpy as jnp
from jax import lax
from jax.experimental import pallas as pl
from jax.experimental.pallas import tpu as pltpu
from jax.experimental.pallas import tpu_sc as plsc

NC = 2
NS = 16
LANES = 16
NW = NC * NS
NPIECE = 8


def _align8(n):
    return ((n + 7) // 8) * 8


def _table_layout(E):
    t3 = E * E * E
    poff = _align8(t3)
    soff = _align8(poff + E * E)
    stride = _align8(soff + E)
    return poff, soff, stride


@functools.partial(jax.jit, static_argnums=(2, 3, 4, 5))
def _embedding_bag_mean(x_words, tbl_flat, B, L, E, D):
    chunk = B // NW
    nwords = L // 4
    nblk = nwords // 3
    POFF, SOFF, STRIDE = _table_layout(E)

    nrem_w = nwords % 3
    rem_bytes = [(3 * nblk + w, j) for w in range(nrem_w) for j in range(4)]
    rem_trips = [tuple(rem_bytes[i: i + 3])
                 for i in range(0, len(rem_bytes) - 2, 3)]
    rem2 = rem_bytes[3 * len(rem_trips):]
    rem_pairs = [tuple(rem2[i: i + 2]) for i in range(0, len(rem2) - 1, 2)]
    rem_sing = rem2[2 * len(rem_pairs):]

    mesh = plsc.VectorSubcoreMesh(core_axis_name="c", subcore_axis_name="s")

    @functools.partial(
        pl.kernel,
        out_type=jax.ShapeDtypeStruct((B, D), jnp.float32),
        mesh=mesh,
        scratch_types=[
            pltpu.VMEM((chunk * nwords,), jnp.int32),
            pltpu.VMEM((chunk, D), jnp.float32),
            pltpu.VMEM((D * STRIDE,), jnp.float32),
            pltpu.SemaphoreType.DMA,
            pltpu.SemaphoreType.DMA,
        ],
        compiler_params=pltpu.CompilerParams(needs_layout_passes=False),
    )
    def sc_kernel(x_hbm, tbl_hbm, out_hbm, x_v, out_v, tbl_v, sem_t, sem_x):
        wid = lax.axis_index("s") * NC + lax.axis_index("c")
        base = wid * chunk
        wbase = base * nwords
        pw = chunk * nwords // NPIECE

        tcopy = pltpu.make_async_copy(tbl_hbm, tbl_v, sem_t)
        tcopy.start()
        pieces = [
            pltpu.make_async_copy(
                x_hbm.at[pl.ds(wbase + r * pw, pw)],
                x_v.at[pl.ds(r * pw, pw)],
                sem_x,
            )
            for r in range(NPIECE)
        ]
        for c in pieces:
            c.start()
        tcopy.wait()

        tsub = [tbl_v.at[pl.ds(d * STRIDE, E * E * E)] for d in range(D)]
        psub = [tbl_v.at[pl.ds(d * STRIDE + POFF, E * E)] for d in range(D)]
        ssub = [tbl_v.at[pl.ds(d * STRIDE + SOFF, E)] for d in range(D)]

        lane = lax.iota(jnp.int32, LANES)
        scale = jnp.float32(1.0 / L)
        e_vec = jnp.full((LANES,), E, jnp.int32)
        dim_idx = [jnp.full((LANES,), d, jnp.int32) for d in range(D)]
        gpp = chunk // NPIECE // LANES

        def peel(word, j):
            sh = 8 * j
            return word >> 24 if sh == 24 else (word >> sh) & 255

        def group_body(g, _):
            rows = g * LANES + lane
            rows_w = rows * nwords

            def blk_body(k, accs):
                accs = list(accs)
                wb = 3 * k
                w0 = plsc.load_gather(x_v, [rows_w + wb])
                w1 = plsc.load_gather(x_v, [rows_w + (wb + 1)])
                w2 = plsc.load_gather(x_v, [rows_w + (wb + 2)])
                byts = [peel(w, j) for w in (w0, w1, w2) for j in range(4)]
                for t in range(4):
                    tidx = (byts[3 * t] * e_vec + byts[3 * t + 1]) * e_vec \
                        + byts[3 * t + 2]
                    for d in range(D):
                        accs[d] = accs[d] + plsc.load_gather(tsub[d], [tidx])
                return tuple(accs)

            accs = lax.fori_loop(
                0, nblk, blk_body,
                tuple(jnp.zeros((LANES,), jnp.float32) for _ in range(D)),
            )
            accs = list(accs)

            if rem_bytes:
                wv = {
                    wd: plsc.load_gather(x_v, [rows_w + wd])
                    for wd in sorted({w for (w, _) in rem_bytes})
                }
                for (a, b, c) in rem_trips:
                    tidx = (peel(wv[a[0]], a[1]) * e_vec
                            + peel(wv[b[0]], b[1])) * e_vec \
                        + peel(wv[c[0]], c[1])
                    for d in range(D):
                        accs[d] = accs[d] + plsc.load_gather(tsub[d], [tidx])
                for (a, b) in rem_pairs:
                    pidx = peel(wv[a[0]], a[1]) * e_vec + peel(wv[b[0]], b[1])
                    for d in range(D):
                        accs[d] = accs[d] + plsc.load_gather(psub[d], [pidx])
                for a in rem_sing:
                    sidx = peel(wv[a[0]], a[1])
                    for d in range(D):
                        accs[d] = accs[d] + plsc.load_gather(ssub[d], [sidx])

            for d in range(D):
                plsc.store_scatter(out_v, [rows, dim_idx[d]], accs[d] * scale)
            return 0

        for r in range(NPIECE):
            pieces[r].wait()
            lax.fori_loop(r * gpp, (r + 1) * gpp, group_body, 0)

        pltpu.sync_copy(out_v, out_hbm.at[pl.ds(base, chunk)])

    return sc_kernel(x_words, tbl_flat)


def _table_mask(E):
    poff, soff, stride = _table_layout(E)
    M = np.zeros((E, stride), np.float32)
    for i in range(E):
        for j in range(E):
            for k in range(E):
                r = (i * E + j) * E + k
                M[i, r] += 1.0
                M[j, r] += 1.0
                M[k, r] += 1.0
            r = poff + i * E + j
            M[i, r] += 1.0
            M[j, r] += 1.0
        M[i, soff + i] += 1.0
    return M


def kernel(x, weight):
    B, L = x.shape
    E, D = weight.shape
    xr = x.astype(jnp.int32).reshape(B, 4, L // 4)
    x_words = (
        xr[:, 0, :] + xr[:, 1, :] * 256
        + xr[:, 2, :] * 65536 + xr[:, 3, :] * 16777216
    ).reshape(B * (L // 4))
    w = weight.astype(jnp.float32)
    tbl = jnp.einsum(
        "ed,es->ds", w, jnp.asarray(_table_mask(E)),
        precision=jax.lax.Precision.HIGHEST,
    ).reshape(-1)
    return _embedding_bag_mean(x_words, tbl, B, L, E, D)
